# baseline (speedup 1.0000x reference)
import functools

import jax
import jax.numpy as jnp
from jax import lax
from jax.experimental import pallas as pl
from jax.experimental.pallas import tpu as pltpu
from jax.experimental.pallas import tpu_sc as plsc

N = 10000
E = 320000
D = 128
G = 64
NB = 10
BLK = N // NB

NCORES = 2
NSUB = 16
NTILES = NCORES * NSUB
CL = 100
CH = 100
HS = (48, 52)
HMAX = max(HS)
NP = 10112
ZR = NP // NSUB


def _build_sc_agg():
    mesh = plsc.VectorSubcoreMesh(core_axis_name="c", subcore_axis_name="s")

    @functools.partial(
        pl.kernel,
        out_type=jax.ShapeDtypeStruct((NCORES, NP, D), jnp.float32),
        mesh=mesh,
        scratch_types=[
            pltpu.VMEM((HMAX, CL), jnp.int32),
            pltpu.VMEM((HMAX, CL), jnp.int32),
            [pltpu.VMEM((CL, D), jnp.float32) for _ in range(2)],
            pltpu.VMEM_SHARED((NP, D), jnp.float32),
            [pltpu.SemaphoreType.DMA for _ in range(4)],
        ],
    )
    def agg(h_hbm, src_hbm, dst_hbm, zero_hbm, out_hbm,
            src_v, dst_v, bufs, acc, sems):
        cid = lax.axis_index("c")
        sid = lax.axis_index("s")
        semg = sems[:2]
        sems_ = sems[2:]
        pltpu.sync_copy(zero_hbm, acc.at[pl.ds(sid * ZR, ZR)])

        def wait_gather(b, l):
            pltpu.make_async_copy(
                h_hbm.at[src_v.at[l]], bufs[b], semg[b]).wait()

        def wait_scatter(b, l):
            pltpu.make_async_copy(
                bufs[b], acc.at[dst_v.at[l]], sems_[b]).wait()

        def issue_gather(b, l):
            pltpu.async_copy(h_hbm.at[src_v.at[l]], bufs[b], semg[b])

        def issue_scatter(b, l):
            pltpu.async_copy(bufs[b], acc.at[dst_v.at[l]], sems_[b],
                             add=True)

        off = 0
        for nc in HS:
            pltpu.sync_copy(src_hbm.at[cid].at[sid].at[pl.ds(off, nc)],
                            src_v.at[pl.ds(0, nc)])
            pltpu.sync_copy(dst_hbm.at[cid].at[sid].at[pl.ds(off, nc)],
                            dst_v.at[pl.ds(0, nc)])
            if off == 0:
                plsc.subcore_barrier()
            for b in range(2):
                issue_gather(b, b)

            def body(p, carry):
                base = 2 * p
                for b in range(2):
                    wait_gather(b, base + b)
                for b in range(2):
                    issue_scatter(b, base + b)
                for b in range(2):
                    wait_scatter(b, base + b)
                for b in range(2):
                    issue_gather(b, base + 2 + b)
                return carry

            lax.fori_loop(0, nc // 2 - 1, body, 0)

            for b in range(2):
                wait_gather(b, nc - 2 + b)
            for b in range(2):
                issue_scatter(b, nc - 2 + b)
            for b in range(2):
                wait_scatter(b, nc - 2 + b)
            off += nc
        plsc.subcore_barrier()
        pltpu.sync_copy(acc.at[pl.ds(sid * ZR, ZR)],
                        out_hbm.at[cid].at[pl.ds(sid * ZR, ZR)])

    return agg


_sc_agg = _build_sc_agg()


def _mlp_stats_body(h_ref, p_ref, w1_ref, b1_ref, w2_ref, b2_ref,
                    u_ref, st_ref):
    i = pl.program_id(0)

    @pl.when(i == 0)
    def _():
        st_ref[...] = jnp.zeros_like(st_ref)

    p = p_ref[...]
    hin = h_ref[...] + p[0] + p[1]
    a = jnp.maximum(
        jnp.dot(hin, w1_ref[...], preferred_element_type=jnp.float32)
        + b1_ref[...], 0.0)
    u = jnp.dot(a, w2_ref[...], preferred_element_type=jnp.float32) + b2_ref[...]
    u_ref[...] = u
    s = jnp.sum(u, axis=0, keepdims=True)
    s2 = jnp.sum(u * u, axis=0, keepdims=True)
    st_ref[...] += jnp.concatenate(
        [s, s2, jnp.zeros((6, D), jnp.float32)], axis=0)


def _mlp_stats(h, parts, w1, b1, w2, b2):
    return pl.pallas_call(
        _mlp_stats_body,
        grid=(NB,),
        in_specs=[
            pl.BlockSpec((BLK, D), lambda i: (i, 0)),
            pl.BlockSpec((NCORES, BLK, D), lambda i: (0, i, 0)),
            pl.BlockSpec((D, D), lambda i: (0, 0)),
            pl.BlockSpec((1, D), lambda i: (0, 0)),
            pl.BlockSpec((D, D), lambda i: (0, 0)),
            pl.BlockSpec((1, D), lambda i: (0, 0)),
        ],
        out_specs=[
            pl.BlockSpec((BLK, D), lambda i: (i, 0)),
            pl.BlockSpec((8, D), lambda i: (0, 0)),
        ],
        out_shape=[
            jax.ShapeDtypeStruct((N, D), jnp.float32),
            jax.ShapeDtypeStruct((8, D), jnp.float32),
        ],
    )(h, parts, w1, b1, w2, b2)


def _bn_relu_body(u_ref, st_ref, g_ref, b_ref, o_ref):
    st = st_ref[...]
    mean = st[0:1] / N
    var = st[1:2] / N - mean * mean
    rstd = lax.rsqrt(var + 1e-5)
    o_ref[...] = jnp.maximum(
        (u_ref[...] - mean) * rstd * g_ref[...] + b_ref[...], 0.0)


def _bn_relu(u, st, gamma, beta):
    return pl.pallas_call(
        _bn_relu_body,
        grid=(NB,),
        in_specs=[
            pl.BlockSpec((BLK, D), lambda i: (i, 0)),
            pl.BlockSpec((8, D), lambda i: (0, 0)),
            pl.BlockSpec((1, D), lambda i: (0, 0)),
            pl.BlockSpec((1, D), lambda i: (0, 0)),
        ],
        out_specs=pl.BlockSpec((BLK, D), lambda i: (i, 0)),
        out_shape=jax.ShapeDtypeStruct((N, D), jnp.float32),
    )(u, st, gamma, beta)


def _final_body(h_ref, p_ref, w1_ref, b1_ref, w2_ref, b2_ref, bt_ref,
                wp1_ref, bp1_ref, wp2_ref, bp2_ref, out_ref, emb_ref):
    i = pl.program_id(0)

    @pl.when(i == 0)
    def _():
        emb_ref[...] = jnp.zeros_like(emb_ref)

    p = p_ref[...]
    hin = h_ref[...] + p[0] + p[1]
    a = jnp.maximum(
        jnp.dot(hin, w1_ref[...], preferred_element_type=jnp.float32)
        + b1_ref[...], 0.0)
    h3 = jnp.dot(a, w2_ref[...], preferred_element_type=jnp.float32) + b2_ref[...]
    bt = bt_ref[...]
    onehot = (lax.broadcasted_iota(jnp.int32, (G, BLK), 0)
              == jnp.broadcast_to(bt[0], (G, BLK))).astype(jnp.float32)
    emb_ref[...] += jnp.dot(onehot, h3, preferred_element_type=jnp.float32)

    @pl.when(i == NB - 1)
    def _():
        pooled = emb_ref[...]
        o1 = jnp.maximum(
            jnp.dot(pooled, wp1_ref[...], preferred_element_type=jnp.float32)
            + bp1_ref[...], 0.0)
        out_ref[...] = (
            jnp.dot(o1, wp2_ref[...], preferred_element_type=jnp.float32)
            + bp2_ref[...])


def _final(h, parts, w1, b1, w2, b2, bt3, wp1, bp1, wp2, bp2):
    return pl.pallas_call(
        _final_body,
        grid=(NB,),
        in_specs=[
            pl.BlockSpec((BLK, D), lambda i: (i, 0)),
            pl.BlockSpec((NCORES, BLK, D), lambda i: (0, i, 0)),
            pl.BlockSpec((D, D), lambda i: (0, 0)),
            pl.BlockSpec((1, D), lambda i: (0, 0)),
            pl.BlockSpec((D, D), lambda i: (0, 0)),
            pl.BlockSpec((1, D), lambda i: (0, 0)),
            pl.BlockSpec((1, 1, BLK), lambda i: (i, 0, 0)),
            pl.BlockSpec((D, D), lambda i: (0, 0)),
            pl.BlockSpec((1, D), lambda i: (0, 0)),
            pl.BlockSpec((D, D), lambda i: (0, 0)),
            pl.BlockSpec((1, D), lambda i: (0, 0)),
        ],
        out_specs=[
            pl.BlockSpec((G, D), lambda i: (0, 0)),
            pl.BlockSpec((G, D), lambda i: (0, 0)),
        ],
        out_shape=[
            jax.ShapeDtypeStruct((G, D), jnp.float32),
            jax.ShapeDtypeStruct((G, D), jnp.float32),
        ],
    )(h, parts, w1, b1, w2, b2, bt3, wp1, bp1, wp2, bp2)


def kernel(x, edge_index, batch,
           W1_0, b1_0, W2_0, b2_0,
           W1_1, b1_1, W2_1, b2_1,
           W1_2, b1_2, W2_2, b2_2,
           gamma_0, beta_0, gamma_1, beta_1,
           Wp1, bp1, Wp2, bp2):
    src2 = edge_index[0].reshape(NCORES, NSUB, CH, CL)
    dst2 = edge_index[1].reshape(NCORES, NSUB, CH, CL)
    zstripe = jnp.zeros((ZR, D), jnp.float32)
    bt3 = batch.reshape(NB, 1, BLK)

    convs = [(W1_0, b1_0.reshape(1, D), W2_0, b2_0.reshape(1, D)),
             (W1_1, b1_1.reshape(1, D), W2_1, b2_1.reshape(1, D)),
             (W1_2, b1_2.reshape(1, D), W2_2, b2_2.reshape(1, D))]
    bns = [(gamma_0.reshape(1, D), beta_0.reshape(1, D)),
           (gamma_1.reshape(1, D), beta_1.reshape(1, D))]

    h = x
    for i in range(2):
        parts = _sc_agg(h, src2, dst2, zstripe)
        w1, b1, w2, b2 = convs[i]
        u, st = _mlp_stats(h, parts, w1, b1, w2, b2)
        g, bt = bns[i]
        h = _bn_relu(u, st, g, bt)
    parts = _sc_agg(h, src2, dst2, zstripe)
    w1, b1, w2, b2 = convs[2]
    out, emb = _final(h, parts, w1, b1, w2, b2, bt3,
                      Wp1, bp1.reshape(1, D), Wp2, bp2.reshape(1, D))
    return (out, emb)


# speedup vs baseline: 1.3035x; 1.3035x over previous
import functools

import jax
import jax.numpy as jnp
from jax import lax
from jax.experimental import pallas as pl
from jax.experimental.pallas import tpu as pltpu
from jax.experimental.pallas import tpu_sc as plsc

N = 10000
E = 320000
D = 128
G = 64
NB = 10
BLK = N // NB

NCORES = 2
NSUB = 16
NTILES = NCORES * NSUB
CL = 80
CH = 125
PARTS = (32, 32, 32, 29)
PMAX = max(PARTS)
NP = 10112
ZR = NP // NSUB


def _build_sc_agg():
    mesh = plsc.VectorSubcoreMesh(core_axis_name="c", subcore_axis_name="s")

    @functools.partial(
        pl.kernel,
        out_type=jax.ShapeDtypeStruct((NCORES, NP, D), jnp.float32),
        mesh=mesh,
        scratch_types=[
            pltpu.VMEM((PMAX, CL), jnp.int32),
            pltpu.VMEM((PMAX, CL), jnp.int32),
            [pltpu.VMEM((CL, D), jnp.float32) for _ in range(4)],
            pltpu.VMEM_SHARED((NP, D), jnp.float32),
            [pltpu.SemaphoreType.DMA for _ in range(8)],
        ],
    )
    def agg(h_hbm, src_hbm, dst_hbm, zero_hbm, out_hbm,
            src_v, dst_v, bufs, acc, sems):
        cid = lax.axis_index("c")
        sid = lax.axis_index("s")
        semg = sems[:4]
        sems_ = sems[4:]
        pltpu.sync_copy(zero_hbm, acc.at[pl.ds(sid * ZR, ZR)])

        def wait_gather(b, l):
            pltpu.make_async_copy(
                h_hbm.at[src_v.at[l]], bufs[b], semg[b]).wait()

        def wait_scatter(b, l):
            pltpu.make_async_copy(
                bufs[b], acc.at[dst_v.at[l]], sems_[b]).wait()

        def issue_gather(b, l):
            pltpu.async_copy(h_hbm.at[src_v.at[l]], bufs[b], semg[b])

        def issue_scatter(b, l):
            pltpu.async_copy(bufs[b], acc.at[dst_v.at[l]], sems_[b],
                             add=True)

        off = 0
        for nc in PARTS:
            q4 = (nc // 4) * 4
            pltpu.sync_copy(src_hbm.at[cid].at[sid].at[pl.ds(off, nc)],
                            src_v.at[pl.ds(0, nc)])
            pltpu.sync_copy(dst_hbm.at[cid].at[sid].at[pl.ds(off, nc)],
                            dst_v.at[pl.ds(0, nc)])
            if off == 0:
                plsc.subcore_barrier()
            issue_gather(0, 0)
            issue_gather(1, 1)
            issue_gather(2, 2)
            wait_gather(0, 0)
            issue_scatter(0, 0)
            issue_gather(3, 3)
            wait_gather(1, 1)
            issue_scatter(1, 1)

            def body(p, carry):
                for j in range(4):
                    l = 4 * p + j
                    wait_scatter(j, l - 4)
                    issue_gather(j, l)
                    wait_gather((j + 2) % 4, l - 2)
                    issue_scatter((j + 2) % 4, l - 2)
                return carry

            lax.fori_loop(1, q4 // 4, body, 0)

            wait_gather((q4 - 2) % 4, q4 - 2)
            issue_scatter((q4 - 2) % 4, q4 - 2)
            wait_gather((q4 - 1) % 4, q4 - 1)
            issue_scatter((q4 - 1) % 4, q4 - 1)
            for l in range(q4 - 4, q4):
                wait_scatter(l % 4, l)
            if nc > q4:
                issue_gather(0, q4)
                wait_gather(0, q4)
                issue_scatter(0, q4)
                wait_scatter(0, q4)
            off += nc
        plsc.subcore_barrier()
        pltpu.sync_copy(acc.at[pl.ds(sid * ZR, ZR)],
                        out_hbm.at[cid].at[pl.ds(sid * ZR, ZR)])

    return agg


_sc_agg = _build_sc_agg()


def _mlp_stats_body(h_ref, p_ref, w1_ref, b1_ref, w2_ref, b2_ref,
                    u_ref, st_ref):
    i = pl.program_id(0)

    @pl.when(i == 0)
    def _():
        st_ref[...] = jnp.zeros_like(st_ref)

    p = p_ref[...]
    hin = h_ref[...] + p[0] + p[1]
    a = jnp.maximum(
        jnp.dot(hin, w1_ref[...], preferred_element_type=jnp.float32)
        + b1_ref[...], 0.0)
    u = jnp.dot(a, w2_ref[...], preferred_element_type=jnp.float32) + b2_ref[...]
    u_ref[...] = u
    s = jnp.sum(u, axis=0, keepdims=True)
    s2 = jnp.sum(u * u, axis=0, keepdims=True)
    st_ref[...] += jnp.concatenate(
        [s, s2, jnp.zeros((6, D), jnp.float32)], axis=0)


def _mlp_stats(h, parts, w1, b1, w2, b2):
    return pl.pallas_call(
        _mlp_stats_body,
        grid=(NB,),
        in_specs=[
            pl.BlockSpec((BLK, D), lambda i: (i, 0)),
            pl.BlockSpec((NCORES, BLK, D), lambda i: (0, i, 0)),
            pl.BlockSpec((D, D), lambda i: (0, 0)),
            pl.BlockSpec((1, D), lambda i: (0, 0)),
            pl.BlockSpec((D, D), lambda i: (0, 0)),
            pl.BlockSpec((1, D), lambda i: (0, 0)),
        ],
        out_specs=[
            pl.BlockSpec((BLK, D), lambda i: (i, 0)),
            pl.BlockSpec((8, D), lambda i: (0, 0)),
        ],
        out_shape=[
            jax.ShapeDtypeStruct((N, D), jnp.float32),
            jax.ShapeDtypeStruct((8, D), jnp.float32),
        ],
    )(h, parts, w1, b1, w2, b2)


def _bn_relu_body(u_ref, st_ref, g_ref, b_ref, o_ref):
    st = st_ref[...]
    mean = st[0:1] / N
    var = st[1:2] / N - mean * mean
    rstd = lax.rsqrt(var + 1e-5)
    o_ref[...] = jnp.maximum(
        (u_ref[...] - mean) * rstd * g_ref[...] + b_ref[...], 0.0)


def _bn_relu(u, st, gamma, beta):
    return pl.pallas_call(
        _bn_relu_body,
        grid=(NB,),
        in_specs=[
            pl.BlockSpec((BLK, D), lambda i: (i, 0)),
            pl.BlockSpec((8, D), lambda i: (0, 0)),
            pl.BlockSpec((1, D), lambda i: (0, 0)),
            pl.BlockSpec((1, D), lambda i: (0, 0)),
        ],
        out_specs=pl.BlockSpec((BLK, D), lambda i: (i, 0)),
        out_shape=jax.ShapeDtypeStruct((N, D), jnp.float32),
    )(u, st, gamma, beta)


def _final_body(h_ref, p_ref, w1_ref, b1_ref, w2_ref, b2_ref, bt_ref,
                wp1_ref, bp1_ref, wp2_ref, bp2_ref, out_ref, emb_ref):
    i = pl.program_id(0)

    @pl.when(i == 0)
    def _():
        emb_ref[...] = jnp.zeros_like(emb_ref)

    p = p_ref[...]
    hin = h_ref[...] + p[0] + p[1]
    a = jnp.maximum(
        jnp.dot(hin, w1_ref[...], preferred_element_type=jnp.float32)
        + b1_ref[...], 0.0)
    h3 = jnp.dot(a, w2_ref[...], preferred_element_type=jnp.float32) + b2_ref[...]
    bt = bt_ref[...]
    onehot = (lax.broadcasted_iota(jnp.int32, (G, BLK), 0)
              == jnp.broadcast_to(bt[0], (G, BLK))).astype(jnp.float32)
    emb_ref[...] += jnp.dot(onehot, h3, preferred_element_type=jnp.float32)

    @pl.when(i == NB - 1)
    def _():
        pooled = emb_ref[...]
        o1 = jnp.maximum(
            jnp.dot(pooled, wp1_ref[...], preferred_element_type=jnp.float32)
            + bp1_ref[...], 0.0)
        out_ref[...] = (
            jnp.dot(o1, wp2_ref[...], preferred_element_type=jnp.float32)
            + bp2_ref[...])


def _final(h, parts, w1, b1, w2, b2, bt3, wp1, bp1, wp2, bp2):
    return pl.pallas_call(
        _final_body,
        grid=(NB,),
        in_specs=[
            pl.BlockSpec((BLK, D), lambda i: (i, 0)),
            pl.BlockSpec((NCORES, BLK, D), lambda i: (0, i, 0)),
            pl.BlockSpec((D, D), lambda i: (0, 0)),
            pl.BlockSpec((1, D), lambda i: (0, 0)),
            pl.BlockSpec((D, D), lambda i: (0, 0)),
            pl.BlockSpec((1, D), lambda i: (0, 0)),
            pl.BlockSpec((1, 1, BLK), lambda i: (i, 0, 0)),
            pl.BlockSpec((D, D), lambda i: (0, 0)),
            pl.BlockSpec((1, D), lambda i: (0, 0)),
            pl.BlockSpec((D, D), lambda i: (0, 0)),
            pl.BlockSpec((1, D), lambda i: (0, 0)),
        ],
        out_specs=[
            pl.BlockSpec((G, D), lambda i: (0, 0)),
            pl.BlockSpec((G, D), lambda i: (0, 0)),
        ],
        out_shape=[
            jax.ShapeDtypeStruct((G, D), jnp.float32),
            jax.ShapeDtypeStruct((G, D), jnp.float32),
        ],
    )(h, parts, w1, b1, w2, b2, bt3, wp1, bp1, wp2, bp2)


def kernel(x, edge_index, batch,
           W1_0, b1_0, W2_0, b2_0,
           W1_1, b1_1, W2_1, b2_1,
           W1_2, b1_2, W2_2, b2_2,
           gamma_0, beta_0, gamma_1, beta_1,
           Wp1, bp1, Wp2, bp2):
    src2 = edge_index[0].reshape(NCORES, NSUB, CH, CL)
    dst2 = edge_index[1].reshape(NCORES, NSUB, CH, CL)
    zstripe = jnp.zeros((ZR, D), jnp.float32)
    bt3 = batch.reshape(NB, 1, BLK)

    convs = [(W1_0, b1_0.reshape(1, D), W2_0, b2_0.reshape(1, D)),
             (W1_1, b1_1.reshape(1, D), W2_1, b2_1.reshape(1, D)),
             (W1_2, b1_2.reshape(1, D), W2_2, b2_2.reshape(1, D))]
    bns = [(gamma_0.reshape(1, D), beta_0.reshape(1, D)),
           (gamma_1.reshape(1, D), beta_1.reshape(1, D))]

    h = x
    for i in range(2):
        parts = _sc_agg(h, src2, dst2, zstripe)
        w1, b1, w2, b2 = convs[i]
        u, st = _mlp_stats(h, parts, w1, b1, w2, b2)
        g, bt = bns[i]
        h = _bn_relu(u, st, g, bt)
    parts = _sc_agg(h, src2, dst2, zstripe)
    w1, b1, w2, b2 = convs[2]
    out, emb = _final(h, parts, w1, b1, w2, b2, bt3,
                      Wp1, bp1.reshape(1, D), Wp2, bp2.reshape(1, D))
    return (out, emb)
